# baseline (speedup 1.0000x reference)
import functools
import math

import jax
import jax.numpy as jnp
from jax import lax
from jax.experimental import pallas as pl
from jax.experimental.pallas import tpu as pltpu

_B, _C, _H, _W = 4, 19, 512, 512
_N = _B * _H * _W
_K = 100000
_THRESH = 0.7
_HB = 128
_GRID = (_B, _H // _HB)


_RT = 8


def _softmax_ce_tile(preds_ref, targets_ref, r0):
    rs = pl.ds(r0, _RT)
    m = preds_ref[0, 0, rs]
    tmax = targets_ref[0, 0, rs]
    psel = m
    for c in range(1, _C):
        pc = preds_ref[0, c, rs]
        m = jnp.maximum(m, pc)
        tc = targets_ref[0, c, rs]
        upd = tc > tmax
        psel = jnp.where(upd, pc, psel)
        tmax = jnp.where(upd, tc, tmax)
    s = jnp.zeros_like(m)
    for c in range(_C):
        s = s + jnp.exp(preds_ref[0, c, rs] - m)
    inv_s = 1.0 / s
    pred_t = jnp.exp(psel - m) * inv_s
    z = jnp.zeros_like(m)
    for c in range(_C):
        z = z + jnp.exp(jnp.exp(preds_ref[0, c, rs] - m) * inv_s)
    loss = jnp.log(z) - pred_t
    return pred_t, loss


_NBLK = _GRID[0] * _GRID[1]
_TC_BLOCKS = 13


def _fast_body(preds_ref, targets_ref, out_ref, sum_acc, cnt_acc):
    step = pl.program_id(0)

    @pl.when(step == 0)
    def _():
        sum_acc[...] = jnp.zeros_like(sum_acc)
        cnt_acc[...] = jnp.zeros_like(cnt_acc)

    ws = jnp.zeros((_RT, _W), jnp.float32)
    cs = jnp.zeros((_RT, _W), jnp.float32)
    for r0 in range(0, _HB, _RT):
        pred_t, loss = _softmax_ce_tile(preds_ref, targets_ref, r0)
        keep = pred_t < jnp.float32(_THRESH)
        ws = ws + jnp.where(keep, loss, 0.0)
        cs = cs + keep.astype(jnp.float32)
    sum_acc[...] += ws
    cnt_acc[...] += cs

    @pl.when(step == _TC_BLOCKS - 1)
    def _():
        out_ref[0, 0] = jnp.sum(sum_acc[...])
        out_ref[0, 1] = jnp.sum(cnt_acc[...])


def _fast_stats(preds, targets):
    nh = _GRID[1]
    in_spec = pl.BlockSpec((1, _C, _HB, _W), lambda i: (i // nh, 0, i % nh, 0))
    return pl.pallas_call(
        _fast_body,
        grid=(_TC_BLOCKS,),
        in_specs=[in_spec, in_spec],
        out_specs=pl.BlockSpec(memory_space=pltpu.SMEM),
        out_shape=jax.ShapeDtypeStruct((1, 2), jnp.float32),
        scratch_shapes=[
            pltpu.VMEM((8, _W), jnp.float32),
            pltpu.VMEM((8, _W), jnp.float32),
        ],
        compiler_params=pltpu.CompilerParams(
            dimension_semantics=("arbitrary",),
        ),
    )(preds, targets)


_SC_NBLK = _NBLK - _TC_BLOCKS
_SC_PX = _SC_NBLK * _HB * _W
_SC_OFF = _HB * _W
_NWORKER = 32
_PX_PER_TILE = _SC_PX // _NWORKER
_SC_CHUNK = 768
_SC_NCHUNK = _PX_PER_TILE // _SC_CHUNK
_LOG_Z0 = 20.375
_LANES = 16

try:
    from jax.experimental.pallas import tpu_sc as plsc
    _HAVE_SC = True
except ImportError:
    _HAVE_SC = False


def _sc_group(pbuf, tbuf, bi, g, ks, kc):
    rs = pl.ds(g * _LANES, _LANES)
    m = pbuf[bi, 0, rs]
    tmax = tbuf[bi, 0, rs]
    psel = m
    for c in range(1, _C):
        pc = pbuf[bi, c, rs]
        m = jnp.maximum(m, pc)
        tc = tbuf[bi, c, rs]
        upd = tc > tmax
        psel = jnp.where(upd, pc, psel)
        tmax = jnp.where(upd, tc, tmax)
    es = []
    s = jnp.zeros((_LANES,), jnp.float32)
    for c in range(_C):
        e = jnp.exp(pbuf[bi, c, rs] - m)
        es.append(e)
        s = s + e
    inv_s = 1.0 / s
    pred_t = jnp.exp(psel - m) * inv_s
    z = jnp.zeros((_LANES,), jnp.float32)
    for c in range(_C):
        z = z + jnp.exp(es[c] * inv_s)
    u = z * jnp.float32(1.0 / _LOG_Z0) - 1.0
    logz = jnp.float32(math.log(_LOG_Z0)) + u * (
        1.0 + u * (-0.5 + u * jnp.float32(1.0 / 3.0)))
    loss = logz - pred_t
    keep = pred_t < jnp.float32(_THRESH)
    ks = ks + jnp.where(keep, loss, 0.0)
    kc = kc + jnp.where(keep, jnp.float32(1.0), jnp.float32(0.0))
    return ks, kc


def _sc_stats(preds3, targets3):
    mesh = plsc.VectorSubcoreMesh(core_axis_name="c", subcore_axis_name="s")

    @functools.partial(
        pl.kernel,
        out_type=jax.ShapeDtypeStruct((2 * _NWORKER, _LANES), jnp.float32),
        mesh=mesh,
        scratch_types=[
            pltpu.VMEM((2, _C, _SC_CHUNK), jnp.float32),
            pltpu.VMEM((2, _C, _SC_CHUNK), jnp.float32),
            pltpu.VMEM((2, _LANES), jnp.float32),
            pltpu.SemaphoreType.DMA,
            pltpu.SemaphoreType.DMA,
            pltpu.SemaphoreType.DMA,
            pltpu.SemaphoreType.DMA,
        ],
    )
    def sck(preds_hbm, targets_hbm, out_hbm, pbuf, tbuf, obuf, sp0, sp1, st0, st1):
        wid = lax.axis_index("s") * 2 + lax.axis_index("c")
        base = _SC_OFF + wid * _PX_PER_TILE
        psems = [sp0, sp1]
        tsems = [st0, st1]

        def start(k):
            bi = k % 2
            cpy_p = pltpu.async_copy(
                preds_hbm.at[:, pl.ds(base + k * _SC_CHUNK, _SC_CHUNK)],
                pbuf.at[bi], psems[bi])
            cpy_t = pltpu.async_copy(
                targets_hbm.at[:, pl.ds(base + k * _SC_CHUNK, _SC_CHUNK)],
                tbuf.at[bi], tsems[bi])
            return cpy_p, cpy_t

        pending = start(0)
        ks = jnp.zeros((_LANES,), jnp.float32)
        kc = jnp.zeros((_LANES,), jnp.float32)
        for k in range(_SC_NCHUNK):
            bi = k % 2
            pending[0].wait()
            pending[1].wait()
            if k + 1 < _SC_NCHUNK:
                pending = start(k + 1)

            def body(g, carry):
                return _sc_group(pbuf, tbuf, bi, g, *carry)

            ks, kc = lax.fori_loop(0, _SC_CHUNK // _LANES, body, (ks, kc))

        obuf[0] = ks
        obuf[1] = kc
        pltpu.sync_copy(obuf, out_hbm.at[pl.ds(wid * 2, 2)])

    return sck(preds3, targets3)


def _ce_out_body(preds_ref, targets_ref, predt_ref, loss_ref):
    for r0 in range(0, _HB, _RT):
        pred_t, loss = _softmax_ce_tile(preds_ref, targets_ref, r0)
        predt_ref[0, pl.ds(r0, _RT)] = pred_t
        loss_ref[0, pl.ds(r0, _RT)] = loss


_ROWS, _COLS = 1024, 1024
_CH = 32
_NCHUNK = _ROWS // _CH


def _select_body(predt_ref, loss_ref, out_ref):
    def count_le(mid):
        def chunk(i, acc):
            blk = predt_ref[pl.ds(i * _CH, _CH), :]
            bits = lax.bitcast_convert_type(blk, jnp.int32)
            mask = (bits <= mid).astype(jnp.int32)
            part = mask[0:8] + mask[8:16] + mask[16:24] + mask[24:32]
            return acc + part
        acc = lax.fori_loop(0, _NCHUNK, chunk,
                            jnp.zeros((8, _COLS), jnp.int32), unroll=2)
        return jnp.sum(acc)

    def bstep(_, carry):
        lo, hi = carry
        mid = lax.div(lo + hi, jnp.int32(2))
        pred = count_le(mid) >= jnp.int32(_K + 1)
        return jnp.where(pred, lo, mid), jnp.where(pred, mid, hi)

    lo0 = jnp.int32(-1)
    hi0 = jnp.int32(0x3F800000)
    _, hi = lax.fori_loop(0, 30, bstep, (lo0, hi0))
    kth = lax.bitcast_convert_type(hi, jnp.float32)
    thr = jnp.maximum(kth, jnp.float32(_THRESH))

    def acc_chunk(i, carry):
        ksum, kcnt = carry
        pt = predt_ref[pl.ds(i * _CH, _CH), :]
        ls = loss_ref[pl.ds(i * _CH, _CH), :]
        keep = pt < thr
        ls = jnp.where(keep, ls, 0.0)
        cnt = keep.astype(jnp.float32)
        ksum = ksum + (ls[0:8] + ls[8:16] + ls[16:24] + ls[24:32])
        kcnt = kcnt + (cnt[0:8] + cnt[8:16] + cnt[16:24] + cnt[24:32])
        return ksum, kcnt

    z8 = jnp.zeros((8, _COLS), jnp.float32)
    ksum, kcnt = lax.fori_loop(0, _NCHUNK, acc_chunk, (z8, z8), unroll=2)
    out_ref[0, 0] = jnp.sum(ksum) / jnp.sum(kcnt)


def _slow_path(preds, targets):
    in_spec = pl.BlockSpec((1, _C, _HB, _W), lambda b, h: (b, 0, h, 0))
    out_spec = pl.BlockSpec((1, _HB, _W), lambda b, h: (b, h, 0))
    pred_t, loss = pl.pallas_call(
        _ce_out_body,
        grid=_GRID,
        in_specs=[in_spec, in_spec],
        out_specs=[out_spec, out_spec],
        out_shape=[
            jax.ShapeDtypeStruct((_B, _H, _W), jnp.float32),
            jax.ShapeDtypeStruct((_B, _H, _W), jnp.float32),
        ],
        compiler_params=pltpu.CompilerParams(
            dimension_semantics=("parallel", "parallel"),
        ),
    )(preds, targets)
    out = pl.pallas_call(
        _select_body,
        in_specs=[
            pl.BlockSpec((_ROWS, _COLS), lambda: (0, 0)),
            pl.BlockSpec((_ROWS, _COLS), lambda: (0, 0)),
        ],
        out_specs=pl.BlockSpec(memory_space=pltpu.SMEM),
        out_shape=jax.ShapeDtypeStruct((1, 1), jnp.float32),
    )(pred_t.reshape(_ROWS, _COLS), loss.reshape(_ROWS, _COLS))
    return out[0, 0]


@jax.jit
def kernel(preds, targets):
    stats = _fast_stats(preds, targets)
    sc = _sc_stats(preds[3].reshape(_C, _H * _W),
                   targets[3].reshape(_C, _H * _W))
    ksum = stats[0, 0] + jnp.sum(sc[0::2])
    kcnt = stats[0, 1] + jnp.sum(sc[1::2])
    return lax.cond(
        kcnt >= jnp.float32(_K + 1),
        lambda: ksum / kcnt,
        lambda: _slow_path(preds, targets),
    )


# speedup vs baseline: 2.6802x; 2.6802x over previous
import functools

import jax
import jax.numpy as jnp
from jax import lax
from jax.experimental import pallas as pl
from jax.experimental.pallas import tpu as pltpu

_B, _C, _H, _W = 4, 19, 512, 512
_N = _B * _H * _W
_K = 100000
_THRESH = 0.7
_HB = 128
_GRID = (_B, _H // _HB)


_RT = 8


def _softmax_ce_tile(preds_ref, targets_ref, r0):
    rs = pl.ds(r0, _RT)
    m = preds_ref[0, 0, rs]
    tmax = targets_ref[0, 0, rs]
    psel = m
    for c in range(1, _C):
        pc = preds_ref[0, c, rs]
        m = jnp.maximum(m, pc)
        tc = targets_ref[0, c, rs]
        upd = tc > tmax
        psel = jnp.where(upd, pc, psel)
        tmax = jnp.where(upd, tc, tmax)
    s = jnp.zeros_like(m)
    for c in range(_C):
        s = s + jnp.exp(preds_ref[0, c, rs] - m)
    inv_s = 1.0 / s
    pred_t = jnp.exp(psel - m) * inv_s
    z = jnp.zeros_like(m)
    for c in range(_C):
        z = z + jnp.exp(jnp.exp(preds_ref[0, c, rs] - m) * inv_s)
    loss = jnp.log(z) - pred_t
    return pred_t, loss


def _fast_body(preds_ref, targets_ref, out_ref, sum_acc, cnt_acc):
    step = pl.program_id(0) * pl.num_programs(1) + pl.program_id(1)

    @pl.when(step == 0)
    def _():
        sum_acc[...] = jnp.zeros_like(sum_acc)
        cnt_acc[...] = jnp.zeros_like(cnt_acc)

    ws = jnp.zeros((_RT, _W), jnp.float32)
    cs = jnp.zeros((_RT, _W), jnp.float32)
    for r0 in range(0, _HB, _RT):
        pred_t, loss = _softmax_ce_tile(preds_ref, targets_ref, r0)
        keep = pred_t < jnp.float32(_THRESH)
        ws = ws + jnp.where(keep, loss, 0.0)
        cs = cs + keep.astype(jnp.float32)
    sum_acc[...] += ws
    cnt_acc[...] += cs

    @pl.when(step == _GRID[0] * _GRID[1] - 1)
    def _():
        out_ref[0, 0] = jnp.sum(sum_acc[...])
        out_ref[0, 1] = jnp.sum(cnt_acc[...])


def _fast_stats(preds, targets):
    in_spec = pl.BlockSpec((1, _C, _HB, _W), lambda b, h: (b, 0, h, 0))
    return pl.pallas_call(
        _fast_body,
        grid=_GRID,
        in_specs=[in_spec, in_spec],
        out_specs=pl.BlockSpec(memory_space=pltpu.SMEM),
        out_shape=jax.ShapeDtypeStruct((1, 2), jnp.float32),
        scratch_shapes=[
            pltpu.VMEM((8, _W), jnp.float32),
            pltpu.VMEM((8, _W), jnp.float32),
        ],
        compiler_params=pltpu.CompilerParams(
            dimension_semantics=("arbitrary", "arbitrary"),
        ),
    )(preds, targets)


def _ce_out_body(preds_ref, targets_ref, predt_ref, loss_ref):
    for r0 in range(0, _HB, _RT):
        pred_t, loss = _softmax_ce_tile(preds_ref, targets_ref, r0)
        predt_ref[0, pl.ds(r0, _RT)] = pred_t
        loss_ref[0, pl.ds(r0, _RT)] = loss


_ROWS, _COLS = 1024, 1024
_CH = 32
_NCHUNK = _ROWS // _CH


def _select_body(predt_ref, loss_ref, out_ref):
    def count_le(mid):
        def chunk(i, acc):
            blk = predt_ref[pl.ds(i * _CH, _CH), :]
            bits = lax.bitcast_convert_type(blk, jnp.int32)
            mask = (bits <= mid).astype(jnp.int32)
            part = mask[0:8] + mask[8:16] + mask[16:24] + mask[24:32]
            return acc + part
        acc = lax.fori_loop(0, _NCHUNK, chunk,
                            jnp.zeros((8, _COLS), jnp.int32), unroll=2)
        return jnp.sum(acc)

    def bstep(_, carry):
        lo, hi = carry
        mid = lax.div(lo + hi, jnp.int32(2))
        pred = count_le(mid) >= jnp.int32(_K + 1)
        return jnp.where(pred, lo, mid), jnp.where(pred, mid, hi)

    lo0 = jnp.int32(-1)
    hi0 = jnp.int32(0x3F800000)
    _, hi = lax.fori_loop(0, 30, bstep, (lo0, hi0))
    kth = lax.bitcast_convert_type(hi, jnp.float32)
    thr = jnp.maximum(kth, jnp.float32(_THRESH))

    def acc_chunk(i, carry):
        ksum, kcnt = carry
        pt = predt_ref[pl.ds(i * _CH, _CH), :]
        ls = loss_ref[pl.ds(i * _CH, _CH), :]
        keep = pt < thr
        ls = jnp.where(keep, ls, 0.0)
        cnt = keep.astype(jnp.float32)
        ksum = ksum + (ls[0:8] + ls[8:16] + ls[16:24] + ls[24:32])
        kcnt = kcnt + (cnt[0:8] + cnt[8:16] + cnt[16:24] + cnt[24:32])
        return ksum, kcnt

    z8 = jnp.zeros((8, _COLS), jnp.float32)
    ksum, kcnt = lax.fori_loop(0, _NCHUNK, acc_chunk, (z8, z8), unroll=2)
    out_ref[0, 0] = jnp.sum(ksum) / jnp.sum(kcnt)


def _slow_path(preds, targets):
    in_spec = pl.BlockSpec((1, _C, _HB, _W), lambda b, h: (b, 0, h, 0))
    out_spec = pl.BlockSpec((1, _HB, _W), lambda b, h: (b, h, 0))
    pred_t, loss = pl.pallas_call(
        _ce_out_body,
        grid=_GRID,
        in_specs=[in_spec, in_spec],
        out_specs=[out_spec, out_spec],
        out_shape=[
            jax.ShapeDtypeStruct((_B, _H, _W), jnp.float32),
            jax.ShapeDtypeStruct((_B, _H, _W), jnp.float32),
        ],
        compiler_params=pltpu.CompilerParams(
            dimension_semantics=("parallel", "parallel"),
        ),
    )(preds, targets)
    out = pl.pallas_call(
        _select_body,
        in_specs=[
            pl.BlockSpec((_ROWS, _COLS), lambda: (0, 0)),
            pl.BlockSpec((_ROWS, _COLS), lambda: (0, 0)),
        ],
        out_specs=pl.BlockSpec(memory_space=pltpu.SMEM),
        out_shape=jax.ShapeDtypeStruct((1, 1), jnp.float32),
    )(pred_t.reshape(_ROWS, _COLS), loss.reshape(_ROWS, _COLS))
    return out[0, 0]


@jax.jit
def kernel(preds, targets):
    stats = _fast_stats(preds, targets)
    ksum, kcnt = stats[0, 0], stats[0, 1]
    return lax.cond(
        kcnt >= jnp.float32(_K + 1),
        lambda: ksum / kcnt,
        lambda: _slow_path(preds, targets),
    )
